# baseline (speedup 1.0000x reference)
import functools
import jax
import jax.numpy as jnp
from jax import lax
from jax.experimental import pallas as pl
from jax.experimental.pallas import tpu as pltpu
from jax.experimental.pallas import tpu_sc as plsc

_B, _S, _D = 4, 8192, 768
_RANK = 4
_E = 16
_SCALE = 1.0 / _RANK
_F = _RANK * _D

_TS = 4096


def _sc_gather(A2, Bt2, sid):
    mesh = plsc.VectorSubcoreMesh(core_axis_name="c", subcore_axis_name="s")

    @functools.partial(
        pl.kernel,
        mesh=mesh,
        out_type=[
            jax.ShapeDtypeStruct((_B, _F), jnp.float32),
            jax.ShapeDtypeStruct((_B, _F), jnp.float32),
        ],
        scratch_types=[
            pltpu.VMEM((_B,), jnp.int32),
            pltpu.VMEM((_B, _F), jnp.float32),
            pltpu.VMEM((_B, _F), jnp.float32),
            pltpu.SemaphoreType.DMA,
            pltpu.SemaphoreType.DMA,
        ],
    )
    def k(a_hbm, bt_hbm, sid_hbm, outa_hbm, outb_hbm, idx_v, rowsa_v, rowsb_v, sema, semb):
        wid = lax.axis_index("s") * 2 + lax.axis_index("c")

        @pl.when(wid == 0)
        def _():
            pltpu.sync_copy(sid_hbm, idx_v)
            ca = pltpu.async_copy(a_hbm.at[idx_v], rowsa_v, sema)
            cb = pltpu.async_copy(bt_hbm.at[idx_v], rowsb_v, semb)
            ca.wait()
            cb.wait()
            pltpu.sync_copy(rowsa_v, outa_hbm)
            pltpu.sync_copy(rowsb_v, outb_hbm)

    return k(A2, Bt2, sid)


def _fused_kernel(x_ref, Wt_ref, b_ref, A_ref, Bt_ref, out_ref, weff_ref):
    @pl.when(pl.program_id(1) == 0)
    def _build_weff():
        weff_ref[...] = Wt_ref[...] + _SCALE * jnp.dot(
            A_ref[0].T, Bt_ref[0], preferred_element_type=jnp.float32
        )

    out_ref[0] = (
        jnp.dot(x_ref[0], weff_ref[...], preferred_element_type=jnp.float32)
        + b_ref[...]
    )


def kernel(x, subject_id, W, b, lora_A, lora_B):
    Wt = W.T
    Bt = lora_B.transpose(0, 2, 1)
    sid = subject_id.astype(jnp.int32)

    A_sel2, Bt_sel2 = _sc_gather(
        lora_A.reshape(_E, _F), Bt.reshape(_E, _F), sid
    )
    A_sel = A_sel2.reshape(_B, _RANK, _D)
    Bt_sel = Bt_sel2.reshape(_B, _RANK, _D)

    n_s = _S // _TS
    grid_spec = pltpu.PrefetchScalarGridSpec(
        num_scalar_prefetch=0,
        grid=(_B, n_s),
        in_specs=[
            pl.BlockSpec((1, _TS, _D), lambda bb, ss: (bb, ss, 0)),
            pl.BlockSpec((_D, _D), lambda bb, ss: (0, 0)),
            pl.BlockSpec((1, _D), lambda bb, ss: (0, 0)),
            pl.BlockSpec((1, _RANK, _D), lambda bb, ss: (bb, 0, 0)),
            pl.BlockSpec((1, _RANK, _D), lambda bb, ss: (bb, 0, 0)),
        ],
        out_specs=pl.BlockSpec((1, _TS, _D), lambda bb, ss: (bb, ss, 0)),
        scratch_shapes=[pltpu.VMEM((_D, _D), jnp.float32)],
    )

    return pl.pallas_call(
        _fused_kernel,
        grid_spec=grid_spec,
        out_shape=jax.ShapeDtypeStruct((_B, _S, _D), jnp.float32),
        compiler_params=pltpu.CompilerParams(
            dimension_semantics=("arbitrary", "arbitrary"),
            vmem_limit_bytes=100 * 1024 * 1024,
        ),
    )(x, Wt, b.reshape(1, _D), A_sel, Bt_sel)


# speedup vs baseline: 1.2374x; 1.2374x over previous
import jax
import jax.numpy as jnp
from jax.experimental import pallas as pl
from jax.experimental.pallas import tpu as pltpu

_B, _S, _D = 4, 8192, 768
_RANK = 4
_E = 16
_SCALE = 1.0 / _RANK

_TS = 4096


def _fused_kernel(sid_ref, x_ref, Wt_ref, b_ref, A_ref, Bt_ref, out_ref, weff_ref):
    @pl.when(pl.program_id(1) == 0)
    def _build_weff():
        weff_ref[...] = Wt_ref[...].T + _SCALE * jnp.dot(
            A_ref[0].T, Bt_ref[0].T, preferred_element_type=jnp.float32
        )

    out_ref[0] = (
        jnp.dot(x_ref[0], weff_ref[...], preferred_element_type=jnp.float32)
        + b_ref[...]
    )


def kernel(x, subject_id, W, b, lora_A, lora_B):
    sid = subject_id.astype(jnp.int32)
    n_s = _S // _TS

    grid_spec = pltpu.PrefetchScalarGridSpec(
        num_scalar_prefetch=1,
        grid=(_B, n_s),
        in_specs=[
            pl.BlockSpec((1, _TS, _D), lambda bb, ss, sid_ref: (bb, ss, 0)),
            pl.BlockSpec((_D, _D), lambda bb, ss, sid_ref: (0, 0)),
            pl.BlockSpec((1, _D), lambda bb, ss, sid_ref: (0, 0)),
            pl.BlockSpec((1, _RANK, _D), lambda bb, ss, sid_ref: (sid_ref[bb], 0, 0)),
            pl.BlockSpec((1, _D, _RANK), lambda bb, ss, sid_ref: (sid_ref[bb], 0, 0)),
        ],
        out_specs=pl.BlockSpec((1, _TS, _D), lambda bb, ss, sid_ref: (bb, ss, 0)),
        scratch_shapes=[pltpu.VMEM((_D, _D), jnp.float32)],
    )

    return pl.pallas_call(
        _fused_kernel,
        grid_spec=grid_spec,
        out_shape=jax.ShapeDtypeStruct((_B, _S, _D), jnp.float32),
        compiler_params=pltpu.CompilerParams(
            dimension_semantics=("arbitrary", "arbitrary"),
            vmem_limit_bytes=100 * 1024 * 1024,
        ),
    )(sid, x, W, b.reshape(1, _D), lora_A, lora_B)


# speedup vs baseline: 1.3272x; 1.0726x over previous
import jax
import jax.numpy as jnp
from jax.experimental import pallas as pl
from jax.experimental.pallas import tpu as pltpu

_B, _S, _D = 4, 8192, 768
_RANK = 4
_E = 16
_SCALE = 1.0 / _RANK

_TS = 4096


def _fused_kernel(sid_ref, x_ref, W_ref, b_ref, A_ref, Bt_ref, out_ref, weff_ref):
    @pl.when(pl.program_id(1) == 0)
    def _build_weff():
        weff_ref[...] = W_ref[...].T + _SCALE * jnp.dot(
            A_ref[0].T, Bt_ref[0], preferred_element_type=jnp.float32
        )

    out_ref[0] = (
        jnp.dot(x_ref[0], weff_ref[...], preferred_element_type=jnp.float32)
        + b_ref[...]
    )


def kernel(x, subject_id, W, b, lora_A, lora_B):
    Bt = lora_B.transpose(0, 2, 1)
    sid = subject_id.astype(jnp.int32)
    n_s = _S // _TS

    grid_spec = pltpu.PrefetchScalarGridSpec(
        num_scalar_prefetch=1,
        grid=(_B, n_s),
        in_specs=[
            pl.BlockSpec((1, _TS, _D), lambda bb, ss, sid_ref: (bb, ss, 0)),
            pl.BlockSpec((_D, _D), lambda bb, ss, sid_ref: (0, 0)),
            pl.BlockSpec((1, _D), lambda bb, ss, sid_ref: (0, 0)),
            pl.BlockSpec((1, _RANK, _D), lambda bb, ss, sid_ref: (sid_ref[bb], 0, 0)),
            pl.BlockSpec((1, _RANK, _D), lambda bb, ss, sid_ref: (sid_ref[bb], 0, 0)),
        ],
        out_specs=pl.BlockSpec((1, _TS, _D), lambda bb, ss, sid_ref: (bb, ss, 0)),
        scratch_shapes=[pltpu.VMEM((_D, _D), jnp.float32)],
    )

    return pl.pallas_call(
        _fused_kernel,
        grid_spec=grid_spec,
        out_shape=jax.ShapeDtypeStruct((_B, _S, _D), jnp.float32),
        compiler_params=pltpu.CompilerParams(
            dimension_semantics=("arbitrary", "arbitrary"),
            vmem_limit_bytes=100 * 1024 * 1024,
        ),
    )(sid, x, W, b.reshape(1, _D), lora_A, Bt)
